# baseline (device time: 277846 ns/iter reference)
import jax
import jax.numpy as jnp
from jax import lax
from jax.experimental import pallas as pl
from jax.experimental.pallas import tpu as pltpu

T = 2048
D = 4096
VH = 8192

NB = 16
VB = VH // NB
NQ = 4
QB = NB // NQ
GR = T // 2
NG = 2 * QB

R_BLK = 128


def _cast_body(x_ref, o_ref):
    o_ref[...] = x_ref[...].astype(jnp.bfloat16)


def _cast_x(x):
    return pl.pallas_call(
        _cast_body,
        grid=(8,),
        in_specs=[pl.BlockSpec((T // 8, D), lambda i: (i, 0))],
        out_specs=pl.BlockSpec((T // 8, D), lambda i: (i, 0)),
        out_shape=jax.ShapeDtypeStruct((T, D), jnp.bfloat16),
    )(x)


def _fused_body(xb_ref, w_ref, l_ref, r_ref,
                wbuf, wsem, lbuf, lsem, sbuf, ssem,
                sx, rx, sy, ry, sz, rz):
    s = pl.program_id(0)
    my_x = lax.axis_index("x")
    my_y = lax.axis_index("y")
    my_z = lax.axis_index("z")
    px = (1 - my_x, my_y, my_z)
    py = (my_x, 1 - my_y, my_z)
    pz = (my_x, my_y, 1 - my_z)
    q_me = 2 * my_y + my_z
    q_y = 2 * (1 - my_y) + my_z
    q_z = 2 * my_y + (1 - my_z)
    j = (QB * q_me + s) % NB
    cur = s % 2

    def w_dma(step, slot):
        jn = (QB * q_me + step) % NB
        return pltpu.make_async_copy(
            w_ref.at[:, pl.ds(jn * VB, VB)], wbuf.at[slot], wsem.at[slot]
        )

    def gref(q, g):
        return r_ref.at[QB * q + g // 2, pl.ds((g % 2) * GR, GR), :]

    def fwd(dst_ref, src_ref, s_sem, r_sem, peer):
        return pltpu.make_async_remote_copy(
            src_ref=src_ref,
            dst_ref=dst_ref,
            send_sem=s_sem,
            recv_sem=r_sem,
            device_id=peer,
            device_id_type=pl.DeviceIdType.MESH,
        )

    @pl.when(s == 0)
    def _():
        barrier = pltpu.get_barrier_semaphore()
        for p in (px, py, pz):
            pl.semaphore_signal(
                barrier, inc=1, device_id=p,
                device_id_type=pl.DeviceIdType.MESH,
            )
        pl.semaphore_wait(barrier, 3)
        w_dma(0, 0).start()

    @pl.when(s + 1 < NB)
    def _():
        w_dma(s + 1, (s + 1) % 2).start()


    @pl.when((s >= 2) & (s < 2 + NG))
    def _():
        k = s - 2
        g = gref(q_me, k)
        fwd(g, g, sx.at[k], rx.at[k], px).wait_recv()
        fwd(g, g, sy.at[k], ry.at[k], py).start()
        fwd(g, g, sz.at[k], rz.at[k], pz).start()

    @pl.when((s >= 8) & (s < 10))
    def _():
        for h in range(2):
            g = 2 * (s - 8) + h
            gr = gref(q_y, g)
            fwd(gr, gr, sy.at[g], ry.at[g], py).wait_recv()
            fwd(gr, gr, sz.at[NG + g], rz.at[NG + g], pz).start()

    @pl.when((s >= 10) & (s < 12))
    def _():
        for h in range(2):
            g = 4 + 2 * (s - 10) + h
            gr = gref(q_z, g)
            fwd(gr, gr, sz.at[g], rz.at[g], pz).wait_recv()
            fwd(gr, gr, sy.at[NG + g - 4], ry.at[NG + g - 4], py).start()


    w_dma(s, cur).wait()
    blk = jnp.dot(
        xb_ref[...], wbuf[cur].astype(jnp.bfloat16),
        preferred_element_type=jnp.float32,
    ).astype(jnp.bfloat16)

    @pl.when(s < NQ)
    def _():
        sbuf[pl.ds(s, 1)] = blk[None]
        pltpu.make_async_copy(sbuf.at[s], l_ref.at[j], ssem.at[s]).start()
        for h in range(2):
            k = 2 * s + h
            fwd(gref(q_me, k), sbuf.at[s, pl.ds(h * GR, GR), :],
                sx.at[k], rx.at[k], px).start()

    @pl.when(s >= NQ)
    def _():
        @pl.when(s >= NQ + 2)
        def _():
            pltpu.make_async_copy(
                lbuf.at[cur], l_ref.at[j], lsem.at[cur]
            ).wait()

        lbuf[pl.ds(cur, 1)] = blk[None]
        pltpu.make_async_copy(
            lbuf.at[cur], l_ref.at[j], lsem.at[cur]
        ).start()

    @pl.when(s == NB - 1)
    def _():
        q_d = 2 * (1 - my_y) + (1 - my_z)
        for g in range(4):
            gr = gref(q_z, g)
            fwd(gr, gr, sz.at[g], rz.at[g], pz).wait_recv()
            gr = gref(q_y, 4 + g)
            fwd(gr, gr, sy.at[4 + g], ry.at[4 + g], py).wait_recv()
            gr = gref(q_d, g)
            fwd(gr, gr, sz.at[NG + g], rz.at[NG + g], pz).wait_recv()
            gr = gref(q_d, 4 + g)
            fwd(gr, gr, sy.at[NG + g], ry.at[NG + g], py).wait_recv()
        for k in range(NG):
            gr = gref(q_me, k)
            fwd(gr, gr, sx.at[k], rx.at[k], px).wait_send()
            fwd(gr, gr, sy.at[k], ry.at[k], py).wait_send()
            fwd(gr, gr, sz.at[k], rz.at[k], pz).wait_send()
        for g in range(4):
            gr = gref(q_y, g)
            fwd(gr, gr, sz.at[NG + g], rz.at[NG + g], pz).wait_send()
            gr = gref(q_z, 4 + g)
            fwd(gr, gr, sy.at[NG + g], ry.at[NG + g], py).wait_send()
        for k in range(NQ):
            pltpu.make_async_copy(
                sbuf.at[k], l_ref.at[k], ssem.at[k]
            ).wait()
        for i in range(2):
            pltpu.make_async_copy(
                lbuf.at[i], l_ref.at[i], lsem.at[i]
            ).wait()


def _fused_gemm_exchange(xb, W):
    blk3 = jax.ShapeDtypeStruct((NB, T, VB), jnp.bfloat16)
    return pl.pallas_call(
        _fused_body,
        grid=(NB,),
        in_specs=[
            pl.BlockSpec(memory_space=pltpu.MemorySpace.VMEM),
            pl.BlockSpec(memory_space=pl.ANY),
        ],
        out_specs=[
            pl.BlockSpec(memory_space=pl.ANY),
            pl.BlockSpec(memory_space=pl.ANY),
        ],
        out_shape=[blk3, blk3],
        scratch_shapes=[
            pltpu.VMEM((2, D, VB), jnp.float32),
            pltpu.SemaphoreType.DMA((2,)),
            pltpu.VMEM((2, T, VB), jnp.bfloat16),
            pltpu.SemaphoreType.DMA((2,)),
            pltpu.VMEM((NQ, T, VB), jnp.bfloat16),
            pltpu.SemaphoreType.DMA((NQ,)),
            pltpu.SemaphoreType.DMA((NG,)),
            pltpu.SemaphoreType.DMA((NG,)),
            pltpu.SemaphoreType.DMA((NG + 4,)),
            pltpu.SemaphoreType.DMA((NG + 4,)),
            pltpu.SemaphoreType.DMA((NG + 4,)),
            pltpu.SemaphoreType.DMA((NG + 4,)),
        ],
        compiler_params=pltpu.CompilerParams(
            collective_id=0, dimension_semantics=("arbitrary",)
        ),
    )(xb, W)


def _softmax_body(l_ref, r_ref, o_ref):
    my_x = lax.axis_index("x")
    lf = l_ref[...].astype(jnp.float32)
    rf = r_ref[...].astype(jnp.float32)
    m = jnp.maximum(
        lf.max(axis=(0, 2), keepdims=True), rf.max(axis=(0, 2), keepdims=True)
    )
    el = jnp.exp(lf - m)
    er = jnp.exp(rf - m)
    s = el.sum(axis=(0, 2), keepdims=True) + er.sum(axis=(0, 2), keepdims=True)
    el = el / s
    er = er / s

    @pl.when(my_x == 0)
    def _():
        for j in range(NB):
            o_ref[:, j * VB:(j + 1) * VB] = el[j]
            o_ref[:, VH + j * VB:VH + (j + 1) * VB] = er[j]

    @pl.when(my_x != 0)
    def _():
        for j in range(NB):
            o_ref[:, j * VB:(j + 1) * VB] = er[j]
            o_ref[:, VH + j * VB:VH + (j + 1) * VB] = el[j]


def _softmax(L, R):
    return pl.pallas_call(
        _softmax_body,
        grid=(T // R_BLK,),
        in_specs=[
            pl.BlockSpec((NB, R_BLK, VB), lambda i: (0, i, 0)),
            pl.BlockSpec((NB, R_BLK, VB), lambda i: (0, i, 0)),
        ],
        out_specs=pl.BlockSpec((R_BLK, 2 * VH), lambda i: (i, 0)),
        out_shape=jax.ShapeDtypeStruct((T, 2 * VH), jnp.float32),
    )(L, R)


def kernel(x, W):
    xb = _cast_x(x)
    L, R = _fused_gemm_exchange(xb, W)
    return _softmax(L, R)


# device time: 267272 ns/iter; 1.0396x vs baseline; 1.0396x over previous
import jax
import jax.numpy as jnp
from jax import lax
from jax.experimental import pallas as pl
from jax.experimental.pallas import tpu as pltpu

T = 2048
D = 4096
VH = 8192

NB = 16
VB = VH // NB
NQ = 4
QB = NB // NQ
GR = T // 2
NG = 2 * QB

R_BLK = 128


def _cast_body(x_ref, o_ref):
    o_ref[...] = x_ref[...].astype(jnp.bfloat16)


def _cast_x(x):
    return pl.pallas_call(
        _cast_body,
        grid=(8,),
        in_specs=[pl.BlockSpec((T // 8, D), lambda i: (i, 0))],
        out_specs=pl.BlockSpec((T // 8, D), lambda i: (i, 0)),
        out_shape=jax.ShapeDtypeStruct((T, D), jnp.bfloat16),
    )(x)


def _fused_body(xb_ref, w_ref, l_ref, r_ref,
                wbuf, wsem, lbuf, lsem, sbuf, ssem,
                sx, rx, sy, ry, sz, rz):
    s = pl.program_id(0)
    my_x = lax.axis_index("x")
    my_y = lax.axis_index("y")
    my_z = lax.axis_index("z")
    px = (1 - my_x, my_y, my_z)
    py = (my_x, 1 - my_y, my_z)
    pz = (my_x, my_y, 1 - my_z)
    q_me = 2 * my_y + my_z
    q_y = 2 * (1 - my_y) + my_z
    q_z = 2 * my_y + (1 - my_z)
    j = (QB * q_me + s) % NB
    cur = s % 2

    def w_dma(step, slot):
        jn = (QB * q_me + step) % NB
        return pltpu.make_async_copy(
            w_ref.at[:, pl.ds(jn * VB, VB)], wbuf.at[slot], wsem.at[slot]
        )

    def gref(q, g):
        return r_ref.at[QB * q + g // 2, pl.ds((g % 2) * GR, GR), :]

    def fwd(dst_ref, src_ref, s_sem, r_sem, peer):
        return pltpu.make_async_remote_copy(
            src_ref=src_ref,
            dst_ref=dst_ref,
            send_sem=s_sem,
            recv_sem=r_sem,
            device_id=peer,
            device_id_type=pl.DeviceIdType.MESH,
        )

    @pl.when(s == 0)
    def _():
        barrier = pltpu.get_barrier_semaphore()
        for p in (px, py, pz):
            pl.semaphore_signal(
                barrier, inc=1, device_id=p,
                device_id_type=pl.DeviceIdType.MESH,
            )
        pl.semaphore_wait(barrier, 3)
        w_dma(0, 0).start()

    @pl.when(s + 1 < NB)
    def _():
        w_dma(s + 1, (s + 1) % 2).start()


    @pl.when((s >= 2) & (s < 2 + NG))
    def _():
        k = s - 2
        g = gref(q_me, k)
        fwd(g, g, sx.at[k], rx.at[k], px).wait_recv()
        fwd(g, g, sy.at[k], ry.at[k], py).start()
        fwd(g, g, sz.at[k], rz.at[k], pz).start()

    @pl.when((s >= 10) & (s < 12))
    def _():
        for h in range(2):
            g = 2 * (s - 10) + h
            gr = gref(q_y, g)
            fwd(gr, gr, sy.at[g], ry.at[g], py).wait_recv()
            fwd(gr, gr, sz.at[NG + g], rz.at[NG + g], pz).start()

    @pl.when((s >= 12) & (s < 14))
    def _():
        for h in range(2):
            g = 4 + 2 * (s - 12) + h
            gr = gref(q_z, g)
            fwd(gr, gr, sz.at[g], rz.at[g], pz).wait_recv()
            fwd(gr, gr, sy.at[NG + g - 4], ry.at[NG + g - 4], py).start()


    w_dma(s, cur).wait()
    blk = jnp.dot(
        xb_ref[...], wbuf[cur].astype(jnp.bfloat16),
        preferred_element_type=jnp.float32,
    ).astype(jnp.bfloat16)

    @pl.when(s < NQ)
    def _():
        sbuf[pl.ds(s, 1)] = blk[None]
        pltpu.make_async_copy(sbuf.at[s], l_ref.at[j], ssem.at[s]).start()
        for h in range(2):
            k = 2 * s + h
            fwd(gref(q_me, k), sbuf.at[s, pl.ds(h * GR, GR), :],
                sx.at[k], rx.at[k], px).start()

    @pl.when(s >= NQ)
    def _():
        @pl.when(s >= NQ + 2)
        def _():
            pltpu.make_async_copy(
                lbuf.at[cur], l_ref.at[j], lsem.at[cur]
            ).wait()

        lbuf[pl.ds(cur, 1)] = blk[None]
        pltpu.make_async_copy(
            lbuf.at[cur], l_ref.at[j], lsem.at[cur]
        ).start()

    @pl.when(s == NB - 1)
    def _():
        q_d = 2 * (1 - my_y) + (1 - my_z)
        for g in range(4):
            gr = gref(q_z, g)
            fwd(gr, gr, sz.at[g], rz.at[g], pz).wait_recv()
            gr = gref(q_y, 4 + g)
            fwd(gr, gr, sy.at[4 + g], ry.at[4 + g], py).wait_recv()
            gr = gref(q_d, g)
            fwd(gr, gr, sz.at[NG + g], rz.at[NG + g], pz).wait_recv()
            gr = gref(q_d, 4 + g)
            fwd(gr, gr, sy.at[NG + g], ry.at[NG + g], py).wait_recv()
        for k in range(NG):
            gr = gref(q_me, k)
            fwd(gr, gr, sx.at[k], rx.at[k], px).wait_send()
            fwd(gr, gr, sy.at[k], ry.at[k], py).wait_send()
            fwd(gr, gr, sz.at[k], rz.at[k], pz).wait_send()
        for g in range(4):
            gr = gref(q_y, g)
            fwd(gr, gr, sz.at[NG + g], rz.at[NG + g], pz).wait_send()
            gr = gref(q_z, 4 + g)
            fwd(gr, gr, sy.at[NG + g], ry.at[NG + g], py).wait_send()
        for k in range(NQ):
            pltpu.make_async_copy(
                sbuf.at[k], l_ref.at[k], ssem.at[k]
            ).wait()
        for i in range(2):
            pltpu.make_async_copy(
                lbuf.at[i], l_ref.at[i], lsem.at[i]
            ).wait()


def _fused_gemm_exchange(xb, W):
    blk3 = jax.ShapeDtypeStruct((NB, T, VB), jnp.bfloat16)
    return pl.pallas_call(
        _fused_body,
        grid=(NB,),
        in_specs=[
            pl.BlockSpec(memory_space=pltpu.MemorySpace.VMEM),
            pl.BlockSpec(memory_space=pl.ANY),
        ],
        out_specs=[
            pl.BlockSpec(memory_space=pl.ANY),
            pl.BlockSpec(memory_space=pl.ANY),
        ],
        out_shape=[blk3, blk3],
        scratch_shapes=[
            pltpu.VMEM((2, D, VB), jnp.float32),
            pltpu.SemaphoreType.DMA((2,)),
            pltpu.VMEM((2, T, VB), jnp.bfloat16),
            pltpu.SemaphoreType.DMA((2,)),
            pltpu.VMEM((NQ, T, VB), jnp.bfloat16),
            pltpu.SemaphoreType.DMA((NQ,)),
            pltpu.SemaphoreType.DMA((NG,)),
            pltpu.SemaphoreType.DMA((NG,)),
            pltpu.SemaphoreType.DMA((NG + 4,)),
            pltpu.SemaphoreType.DMA((NG + 4,)),
            pltpu.SemaphoreType.DMA((NG + 4,)),
            pltpu.SemaphoreType.DMA((NG + 4,)),
        ],
        compiler_params=pltpu.CompilerParams(
            collective_id=0, dimension_semantics=("arbitrary",)
        ),
    )(xb, W)


def _softmax_body(l_ref, r_ref, o_ref):
    my_x = lax.axis_index("x")
    lf = l_ref[...].astype(jnp.float32)
    rf = r_ref[...].astype(jnp.float32)
    m = jnp.maximum(
        lf.max(axis=(0, 2), keepdims=True), rf.max(axis=(0, 2), keepdims=True)
    )
    el = jnp.exp(lf - m)
    er = jnp.exp(rf - m)
    s = el.sum(axis=(0, 2), keepdims=True) + er.sum(axis=(0, 2), keepdims=True)
    inv = 1.0 / s
    el = el * inv
    er = er * inv

    @pl.when(my_x == 0)
    def _():
        for j in range(NB):
            o_ref[:, j * VB:(j + 1) * VB] = el[j]
            o_ref[:, VH + j * VB:VH + (j + 1) * VB] = er[j]

    @pl.when(my_x != 0)
    def _():
        for j in range(NB):
            o_ref[:, j * VB:(j + 1) * VB] = er[j]
            o_ref[:, VH + j * VB:VH + (j + 1) * VB] = el[j]


def _softmax(L, R):
    return pl.pallas_call(
        _softmax_body,
        grid=(T // R_BLK,),
        in_specs=[
            pl.BlockSpec((NB, R_BLK, VB), lambda i: (0, i, 0)),
            pl.BlockSpec((NB, R_BLK, VB), lambda i: (0, i, 0)),
        ],
        out_specs=pl.BlockSpec((R_BLK, 2 * VH), lambda i: (i, 0)),
        out_shape=jax.ShapeDtypeStruct((T, 2 * VH), jnp.float32),
    )(L, R)


def kernel(x, W):
    xb = _cast_x(x)
    L, R = _fused_gemm_exchange(xb, W)
    return _softmax(L, R)
